# Initial kernel scaffold; baseline (speedup 1.0000x reference)
import jax
import jax.numpy as jnp
from jax.experimental import pallas as pl


def kernel(input, W_off, b_off, W_mask, b_mask):
    raise NotImplementedError("write your pallas kernel here")


# speedup vs baseline: 94.4014x; 94.4014x over previous
import functools
import jax
import jax.numpy as jnp
from jax.experimental import pallas as pl
from jax.experimental.pallas import tpu as pltpu

_CH = 192
_G = 4
_CG = 48
_P = 9
_GP = _G * _P


def _proj_body(x_ref, wo_ref, bo_ref, wm_ref, bm_ref, off_ref, mask_ref):
    x = x_ref[...]
    off_ref[...] = (
        jnp.dot(x, wo_ref[...], preferred_element_type=jnp.float32) + bo_ref[...]
    )
    logits = (
        jnp.dot(x, wm_ref[...], preferred_element_type=jnp.float32) + bm_ref[...]
    )
    logits = logits - jnp.max(logits, axis=-1, keepdims=True)
    e = jnp.exp(logits)
    r = jax.lax.broadcasted_iota(jnp.int32, (_GP, _GP), 0) // _P
    c = jax.lax.broadcasted_iota(jnp.int32, (_GP, _GP), 1) // _P
    a = (r == c).astype(jnp.float32)
    denom = jnp.dot(e, a, preferred_element_type=jnp.float32)
    mask_ref[...] = e / denom


def _projections(x2d, W_off, b_off, W_mask, b_mask):
    n = x2d.shape[0]
    blk = 1568
    grid = n // blk
    return pl.pallas_call(
        _proj_body,
        grid=(grid,),
        in_specs=[
            pl.BlockSpec((blk, _CH), lambda i: (i, 0)),
            pl.BlockSpec((_CH, _GP * 2), lambda i: (0, 0)),
            pl.BlockSpec((1, _GP * 2), lambda i: (0, 0)),
            pl.BlockSpec((_CH, _GP), lambda i: (0, 0)),
            pl.BlockSpec((1, _GP), lambda i: (0, 0)),
        ],
        out_specs=[
            pl.BlockSpec((blk, _GP * 2), lambda i: (i, 0)),
            pl.BlockSpec((blk, _GP), lambda i: (i, 0)),
        ],
        out_shape=[
            jax.ShapeDtypeStruct((n, _GP * 2), jnp.float32),
            jax.ShapeDtypeStruct((n, _GP), jnp.float32),
        ],
    )(x2d, W_off, b_off.reshape(1, -1), W_mask, b_mask.reshape(1, -1))


def kernel(input, W_off, b_off, W_mask, b_mask):
    b, c, h, w = input.shape
    x = input.reshape(b, h, w, c)
    x2d = x.reshape(b * h * w, c)
    offset, mask = _projections(x2d, W_off, b_off, W_mask, b_mask)
    offset = offset.reshape(b, h, w, _GP * 2)
    mask = mask.reshape(b, h, w, _GP)

    H_in, W_in = h + 2, w + 2
    xpad = jnp.pad(x, ((0, 0), (1, 1), (1, 1), (0, 0)))

    lin = jnp.array([-1.0, 0.0, 1.0], jnp.float32)
    dx = jnp.repeat(lin, 3)
    dy = jnp.tile(lin, 3)
    off = offset.reshape(b, h, w, _G, _P, 2)
    ix = (
        jnp.arange(w, dtype=jnp.float32)[None, None, :, None, None]
        + 1.0
        + dx[None, None, None, None, :]
        + off[..., 0]
    )
    iy = (
        jnp.arange(h, dtype=jnp.float32)[None, :, None, None, None]
        + 1.0
        + dy[None, None, None, None, :]
        + off[..., 1]
    )
    ix0 = jnp.floor(ix)
    iy0 = jnp.floor(iy)
    fx = ix - ix0
    fy = iy - iy0

    flatg = xpad.reshape(b, H_in * W_in, _G, _CG)
    mask5 = mask.reshape(b, h, w, _G, _P)
    out = jnp.zeros((b, h, w, _G, _CG), jnp.float32)
    for ddx, ddy, wt in (
        (0, 0, (1.0 - fx) * (1.0 - fy)),
        (1, 0, fx * (1.0 - fy)),
        (0, 1, (1.0 - fx) * fy),
        (1, 1, fx * fy),
    ):
        cx = ix0 + ddx
        cy = iy0 + ddy
        valid = (cx >= 0) & (cx <= W_in - 1) & (cy >= 0) & (cy <= H_in - 1)
        cxc = jnp.clip(cx, 0, W_in - 1).astype(jnp.int32)
        cyc = jnp.clip(cy, 0, H_in - 1).astype(jnp.int32)
        idx = cyc * W_in + cxc
        wv = (wt * valid.astype(jnp.float32)) * mask5
        for g in range(_G):
            idxg = idx[:, :, :, g, :].reshape(b, h * w * _P)
            gathered = jnp.take_along_axis(
                flatg[:, :, g, :],
                idxg[:, :, None],
                axis=1,
            ).reshape(b, h, w, _P, _CG)
            out = out.at[:, :, :, g, :].add(
                (gathered * wv[:, :, :, g, :, None]).sum(3)
            )

    return out.reshape(b, c, h, w)


# speedup vs baseline: 3281.6316x; 34.7625x over previous
import functools
import jax
import jax.numpy as jnp
from jax import lax
from jax.experimental import pallas as pl
from jax.experimental.pallas import tpu as pltpu
from jax.experimental.pallas import tpu_sc as plsc

_CH = 192
_G = 4
_CG = 48
_P = 9
_GP = _G * _P
_H = 56
_W = 56
_HW = _H * _W
_HP = _H + 2
_NPIX_PAD = _HP * _HP
_PC = 224
_NCHUNK = _HW // _PC
_NBLK = _PC // 16
_NC = 2
_NS = 16
_NW = _NC * _NS
_NT = 36


def _proj_body(x_ref, wo_ref, bo_ref, wm_ref, bm_ref, off_ref, mask_ref):
    x = x_ref[...]
    off_ref[...] = (
        jnp.dot(x, wo_ref[...], preferred_element_type=jnp.float32) + bo_ref[...]
    )
    logits = (
        jnp.dot(x, wm_ref[...], preferred_element_type=jnp.float32) + bm_ref[...]
    )
    logits = logits - jnp.max(logits, axis=-1, keepdims=True)
    e = jnp.exp(logits)
    r = jax.lax.broadcasted_iota(jnp.int32, (_GP, _GP), 0) // _P
    c = jax.lax.broadcasted_iota(jnp.int32, (_GP, _GP), 1) // _P
    a = (r == c).astype(jnp.float32)
    denom = jnp.dot(e, a, preferred_element_type=jnp.float32)
    mask_ref[...] = e / denom


def _projections(x2d, W_off, b_off, W_mask, b_mask):
    n = x2d.shape[0]
    blk = 1568
    grid = n // blk
    return pl.pallas_call(
        _proj_body,
        grid=(grid,),
        in_specs=[
            pl.BlockSpec((blk, _CH), lambda i: (i, 0)),
            pl.BlockSpec((_CH, _GP * 2), lambda i: (0, 0)),
            pl.BlockSpec((1, _GP * 2), lambda i: (0, 0)),
            pl.BlockSpec((_CH, _GP), lambda i: (0, 0)),
            pl.BlockSpec((1, _GP), lambda i: (0, 0)),
        ],
        out_specs=[
            pl.BlockSpec((blk, _GP * 2), lambda i: (i, 0)),
            pl.BlockSpec((blk, _GP), lambda i: (i, 0)),
        ],
        out_shape=[
            jax.ShapeDtypeStruct((n, _GP * 2), jnp.float32),
            jax.ShapeDtypeStruct((n, _GP), jnp.float32),
        ],
    )(x2d, W_off, b_off.reshape(1, -1), W_mask, b_mask.reshape(1, -1))


def _idxw_body(ox_ref, oy_ref, m_ref, idx_ref, wts_ref):
    shp = (_P, _HW)
    pix = jax.lax.broadcasted_iota(jnp.int32, shp, 1)
    prow = jax.lax.broadcasted_iota(jnp.int32, shp, 0)
    px = (pix % _W).astype(jnp.float32)
    py = (pix // _W).astype(jnp.float32)
    dx = (prow // 3 - 1).astype(jnp.float32)
    dy = (prow % 3 - 1).astype(jnp.float32)
    ix = px + 1.0 + dx + ox_ref[0, 0]
    iy = py + 1.0 + dy + oy_ref[0, 0]
    ix0 = jnp.floor(ix)
    iy0 = jnp.floor(iy)
    fx = ix - ix0
    fy = iy - iy0
    m = m_ref[0, 0]
    idxs = []
    wts = []
    for adx, ady, wk in (
        (0.0, 0.0, (1.0 - fx) * (1.0 - fy)),
        (1.0, 0.0, fx * (1.0 - fy)),
        (0.0, 1.0, (1.0 - fx) * fy),
        (1.0, 1.0, fx * fy),
    ):
        cx = ix0 + adx
        cy = iy0 + ady
        valid = (
            (cx >= 0.0) & (cx <= _HP - 1.0) & (cy >= 0.0) & (cy <= _HP - 1.0)
        ).astype(jnp.float32)
        cxc = jnp.clip(cx, 0.0, _HP - 1.0).astype(jnp.int32)
        cyc = jnp.clip(cy, 0.0, _HP - 1.0).astype(jnp.int32)
        idxs.append((cyc * _HP + cxc) * 16)
        wts.append(wk * valid * m)
    idx_ref[0, 0] = jnp.concatenate(idxs, axis=0)
    wts_ref[0, 0] = jnp.concatenate(wts, axis=0)


def _idx_weights(offx, offy, mask4):
    return pl.pallas_call(
        _idxw_body,
        grid=(8, _G),
        in_specs=[
            pl.BlockSpec((1, 1, _P, _HW), lambda b, g: (b, g, 0, 0)),
            pl.BlockSpec((1, 1, _P, _HW), lambda b, g: (b, g, 0, 0)),
            pl.BlockSpec((1, 1, _P, _HW), lambda b, g: (b, g, 0, 0)),
        ],
        out_specs=[
            pl.BlockSpec((1, 1, _NT, _HW), lambda b, g: (b, g, 0, 0)),
            pl.BlockSpec((1, 1, _NT, _HW), lambda b, g: (b, g, 0, 0)),
        ],
        out_shape=[
            jax.ShapeDtypeStruct((8, _G, _NT, _HW), jnp.int32),
            jax.ShapeDtypeStruct((8, _G, _NT, _HW), jnp.float32),
        ],
    )(offx, offy, mask4)


def _sc_sample_body(img_hbm, idx_hbm, wts_hbm, out_hbm, img_v, idx_v, wts_v, out_v):
    wid = lax.axis_index("s") * _NC + lax.axis_index("c")
    for k in range(3):
        task = wid * 3 + k
        b = task // 12
        rem = task % 12
        g = rem // 3
        cc = rem % 3
        pltpu.sync_copy(img_hbm.at[b, g * 3 + cc], img_v)

        def chunk_body(c, carry):
            pltpu.sync_copy(idx_hbm.at[b, g, c], idx_v)
            pltpu.sync_copy(wts_hbm.at[b, g, c], wts_v)

            def blk_body(jb, carry2):
                base = jb * 16
                accs = [jnp.zeros((16,), jnp.float32)] * 16
                for t in range(_NT):
                    iv = idx_v[t, pl.ds(base, 16)]
                    wv = wts_v[t, pl.ds(base, 16)]
                    for ch in range(16):
                        gi = iv + ch if ch else iv
                        gv = plsc.load_gather(img_v, [gi])
                        accs[ch] = accs[ch] + gv * wv
                for ch in range(16):
                    out_v[ch, pl.ds(base, 16)] = accs[ch]
                return carry2

            lax.fori_loop(0, _NBLK, blk_body, 0)
            pltpu.sync_copy(out_v, out_hbm.at[b, g, cc, c])
            return carry

        lax.fori_loop(0, _NCHUNK, chunk_body, 0)


def _sc_sample(img, idx_c, wts_c):
    mesh = plsc.VectorSubcoreMesh(core_axis_name="c", subcore_axis_name="s")
    f = functools.partial(
        pl.kernel,
        out_type=jax.ShapeDtypeStruct((8, _G, 3, _NCHUNK, 16, _PC), jnp.float32),
        mesh=mesh,
        scratch_types=[
            pltpu.VMEM((_NPIX_PAD * 16,), jnp.float32),
            pltpu.VMEM((_NT, _PC), jnp.int32),
            pltpu.VMEM((_NT, _PC), jnp.float32),
            pltpu.VMEM((16, _PC), jnp.float32),
        ],
        compiler_params=pltpu.CompilerParams(needs_layout_passes=False),
    )(_sc_sample_body)
    return f(img, idx_c, wts_c)


def kernel(input, W_off, b_off, W_mask, b_mask):
    b, c, h, w = input.shape
    x = input.reshape(b, h, w, c)
    x2d = x.reshape(b * h * w, c)
    offset, mask = _projections(x2d, W_off, b_off, W_mask, b_mask)

    off3 = offset.reshape(b, _HW, _GP, 2)
    offx = off3[..., 0].transpose(0, 2, 1).reshape(b, _G, _P, _HW)
    offy = off3[..., 1].transpose(0, 2, 1).reshape(b, _G, _P, _HW)
    mask4 = mask.reshape(b, _HW, _GP).transpose(0, 2, 1).reshape(b, _G, _P, _HW)

    idx16, wts = _idx_weights(offx, offy, mask4)
    idx_c = idx16.reshape(b, _G, _NT, _NCHUNK, _PC).transpose(0, 1, 3, 2, 4)
    wts_c = wts.reshape(b, _G, _NT, _NCHUNK, _PC).transpose(0, 1, 3, 2, 4)

    xpad = jnp.pad(x, ((0, 0), (1, 1), (1, 1), (0, 0)))
    img = (
        xpad.reshape(b, _NPIX_PAD, 12, 16)
        .transpose(0, 2, 1, 3)
        .reshape(b, 12, _NPIX_PAD * 16)
    )

    out6 = _sc_sample(img, idx_c, wts_c)
    out = out6.transpose(0, 3, 5, 1, 2, 4).reshape(b, _HW, c)
    return out.reshape(b, c, h, w)


# speedup vs baseline: 6037.0850x; 1.8397x over previous
import functools
import jax
import jax.numpy as jnp
from jax import lax
from jax.experimental import pallas as pl
from jax.experimental.pallas import tpu as pltpu
from jax.experimental.pallas import tpu_sc as plsc

_CH = 192
_G = 4
_CG = 48
_P = 9
_GP = _G * _P
_H = 56
_W = 56
_HW = _H * _W
_HP = _H + 2
_NPIX_PAD = _HP * _HP
_PC = 224
_NCHUNK = _HW // _PC
_NBLK = _PC // 16
_NC = 2
_NS = 16
_NW = _NC * _NS
_NT = 36


def _proj_body(x_ref, wo_ref, bo_ref, wm_ref, bm_ref, off_ref, mask_ref):
    x = x_ref[...]
    off_ref[...] = (
        jnp.dot(x, wo_ref[...], preferred_element_type=jnp.float32) + bo_ref[...]
    )
    logits = (
        jnp.dot(x, wm_ref[...], preferred_element_type=jnp.float32) + bm_ref[...]
    )
    logits = logits - jnp.max(logits, axis=-1, keepdims=True)
    e = jnp.exp(logits)
    r = jax.lax.broadcasted_iota(jnp.int32, (_GP, _GP), 0) // _P
    c = jax.lax.broadcasted_iota(jnp.int32, (_GP, _GP), 1) // _P
    a = (r == c).astype(jnp.float32)
    denom = jnp.dot(e, a, preferred_element_type=jnp.float32)
    mask_ref[...] = e / denom


def _projections(x2d, W_off, b_off, W_mask, b_mask):
    n = x2d.shape[0]
    blk = 1568
    grid = n // blk
    return pl.pallas_call(
        _proj_body,
        grid=(grid,),
        in_specs=[
            pl.BlockSpec((blk, _CH), lambda i: (i, 0)),
            pl.BlockSpec((_CH, _GP * 2), lambda i: (0, 0)),
            pl.BlockSpec((1, _GP * 2), lambda i: (0, 0)),
            pl.BlockSpec((_CH, _GP), lambda i: (0, 0)),
            pl.BlockSpec((1, _GP), lambda i: (0, 0)),
        ],
        out_specs=[
            pl.BlockSpec((blk, _GP * 2), lambda i: (i, 0)),
            pl.BlockSpec((blk, _GP), lambda i: (i, 0)),
        ],
        out_shape=[
            jax.ShapeDtypeStruct((n, _GP * 2), jnp.float32),
            jax.ShapeDtypeStruct((n, _GP), jnp.float32),
        ],
    )(x2d, W_off, b_off.reshape(1, -1), W_mask, b_mask.reshape(1, -1))


def _idxw_body(ox_ref, oy_ref, m_ref, idx_ref, wts_ref):
    shp = (_P, _HW)
    pix = jax.lax.broadcasted_iota(jnp.int32, shp, 1)
    prow = jax.lax.broadcasted_iota(jnp.int32, shp, 0)
    px = (pix % _W).astype(jnp.float32)
    py = (pix // _W).astype(jnp.float32)
    dx = (prow // 3 - 1).astype(jnp.float32)
    dy = (prow % 3 - 1).astype(jnp.float32)
    ix = px + 1.0 + dx + ox_ref[0, 0]
    iy = py + 1.0 + dy + oy_ref[0, 0]
    ix0 = jnp.floor(ix)
    iy0 = jnp.floor(iy)
    fx = ix - ix0
    fy = iy - iy0
    m = m_ref[0, 0]
    idxs = []
    wts = []
    for adx, ady, wk in (
        (0.0, 0.0, (1.0 - fx) * (1.0 - fy)),
        (1.0, 0.0, fx * (1.0 - fy)),
        (0.0, 1.0, (1.0 - fx) * fy),
        (1.0, 1.0, fx * fy),
    ):
        cx = ix0 + adx
        cy = iy0 + ady
        valid = (
            (cx >= 0.0) & (cx <= _HP - 1.0) & (cy >= 0.0) & (cy <= _HP - 1.0)
        ).astype(jnp.float32)
        cxc = jnp.clip(cx, 0.0, _HP - 1.0).astype(jnp.int32)
        cyc = jnp.clip(cy, 0.0, _HP - 1.0).astype(jnp.int32)
        idxs.append(cyc * _HP + cxc)
        wts.append(wk * valid * m)
    idx_ref[0, 0] = jnp.concatenate(idxs, axis=0)
    wts_ref[0, 0] = jnp.concatenate(wts, axis=0)


def _idx_weights(offx, offy, mask4):
    return pl.pallas_call(
        _idxw_body,
        grid=(8, _G),
        in_specs=[
            pl.BlockSpec((1, 1, _P, _HW), lambda b, g: (b, g, 0, 0)),
            pl.BlockSpec((1, 1, _P, _HW), lambda b, g: (b, g, 0, 0)),
            pl.BlockSpec((1, 1, _P, _HW), lambda b, g: (b, g, 0, 0)),
        ],
        out_specs=[
            pl.BlockSpec((1, 1, _NT, _HW), lambda b, g: (b, g, 0, 0)),
            pl.BlockSpec((1, 1, _NT, _HW), lambda b, g: (b, g, 0, 0)),
        ],
        out_shape=[
            jax.ShapeDtypeStruct((8, _G, _NT, _HW), jnp.int32),
            jax.ShapeDtypeStruct((8, _G, _NT, _HW), jnp.float32),
        ],
    )(offx, offy, mask4)


def _sc_sample_body(
    img_hbm,
    idx_hbm,
    wts_hbm,
    out_hbm,
    img_v,
    idx_v0,
    idx_v1,
    wts_v0,
    wts_v1,
    out_v0,
    out_v1,
    sem_i0,
    sem_i1,
    sem_w0,
    sem_w1,
    sem_o0,
    sem_o1,
):
    wid = lax.axis_index("s") * _NC + lax.axis_index("c")
    idx_b = [idx_v0, idx_v1]
    wts_b = [wts_v0, wts_v1]
    out_b = [out_v0, out_v1]
    sem_i = [sem_i0, sem_i1]
    sem_w = [sem_w0, sem_w1]
    sem_o = [sem_o0, sem_o1]
    for k in range(3):
        task = wid * 3 + k
        b = task // 12
        rem = task % 12
        g = rem // 3
        cc = rem % 3
        pltpu.sync_copy(img_hbm.at[b, g * 3 + cc], img_v)
        pltpu.async_copy(idx_hbm.at[b, g, :, pl.ds(0, _PC)], idx_b[0], sem_i[0])
        pltpu.async_copy(wts_hbm.at[b, g, :, pl.ds(0, _PC)], wts_b[0], sem_w[0])

        def pair_body(cp, carry):
            for par in range(2):
                c = cp * 2 + par
                nxt = 1 - par

                @pl.when(c + 1 < _NCHUNK)
                def _start_next():
                    pltpu.async_copy(
                        idx_hbm.at[b, g, :, pl.ds((c + 1) * _PC, _PC)],
                        idx_b[nxt],
                        sem_i[nxt],
                    )
                    pltpu.async_copy(
                        wts_hbm.at[b, g, :, pl.ds((c + 1) * _PC, _PC)],
                        wts_b[nxt],
                        sem_w[nxt],
                    )

                pltpu.make_async_copy(
                    idx_hbm.at[b, g, :, pl.ds(c * _PC, _PC)], idx_b[par], sem_i[par]
                ).wait()
                pltpu.make_async_copy(
                    wts_hbm.at[b, g, :, pl.ds(c * _PC, _PC)], wts_b[par], sem_w[par]
                ).wait()

                @pl.when(cp > 0)
                def _wait_out():
                    pltpu.make_async_copy(
                        out_b[par], out_hbm.at[b, g, cc, c - 2], sem_o[par]
                    ).wait()

                idx_v = idx_b[par]
                wts_v = wts_b[par]
                out_v = out_b[par]

                def blk_body(jb, carry2):
                    base = jb * 16
                    accs = [jnp.zeros((16,), jnp.float32)] * 16
                    for t in range(_NT):
                        iv = idx_v[t, pl.ds(base, 16)]
                        wv = wts_v[t, pl.ds(base, 16)]
                        for ch in range(16):
                            gi = iv + (ch * _NPIX_PAD) if ch else iv
                            gv = plsc.load_gather(img_v, [gi])
                            accs[ch] = accs[ch] + gv * wv
                    for ch in range(16):
                        out_v[ch, pl.ds(base, 16)] = accs[ch]
                    return carry2

                lax.fori_loop(0, _NBLK, blk_body, 0)
                pltpu.async_copy(out_v, out_hbm.at[b, g, cc, c], sem_o[par])
            return carry

        lax.fori_loop(0, _NCHUNK // 2, pair_body, 0)
        pltpu.make_async_copy(
            out_b[0], out_hbm.at[b, g, cc, _NCHUNK - 2], sem_o[0]
        ).wait()
        pltpu.make_async_copy(
            out_b[1], out_hbm.at[b, g, cc, _NCHUNK - 1], sem_o[1]
        ).wait()


def _sc_sample(img, idx, wts):
    mesh = plsc.VectorSubcoreMesh(core_axis_name="c", subcore_axis_name="s")
    f = functools.partial(
        pl.kernel,
        out_type=jax.ShapeDtypeStruct((8, _G, 3, _NCHUNK, 16, _PC), jnp.float32),
        mesh=mesh,
        scratch_types=[
            pltpu.VMEM((16 * _NPIX_PAD,), jnp.float32),
            pltpu.VMEM((_NT, _PC), jnp.int32),
            pltpu.VMEM((_NT, _PC), jnp.int32),
            pltpu.VMEM((_NT, _PC), jnp.float32),
            pltpu.VMEM((_NT, _PC), jnp.float32),
            pltpu.VMEM((16, _PC), jnp.float32),
            pltpu.VMEM((16, _PC), jnp.float32),
            pltpu.SemaphoreType.DMA,
            pltpu.SemaphoreType.DMA,
            pltpu.SemaphoreType.DMA,
            pltpu.SemaphoreType.DMA,
            pltpu.SemaphoreType.DMA,
            pltpu.SemaphoreType.DMA,
        ],
        compiler_params=pltpu.CompilerParams(
            needs_layout_passes=False, use_tc_tiling_on_sc=False
        ),
    )(_sc_sample_body)
    return f(img, idx, wts)


def kernel(input, W_off, b_off, W_mask, b_mask):
    b, c, h, w = input.shape
    x = input.reshape(b, h, w, c)
    x2d = x.reshape(b * h * w, c)
    offset, mask = _projections(x2d, W_off, b_off, W_mask, b_mask)

    off3 = offset.reshape(b, _HW, _GP, 2)
    offx = off3[..., 0].transpose(0, 2, 1).reshape(b, _G, _P, _HW)
    offy = off3[..., 1].transpose(0, 2, 1).reshape(b, _G, _P, _HW)
    mask4 = mask.reshape(b, _HW, _GP).transpose(0, 2, 1).reshape(b, _G, _P, _HW)

    idx, wts = _idx_weights(offx, offy, mask4)

    xpad = jnp.pad(x, ((0, 0), (1, 1), (1, 1), (0, 0)))
    img = (
        xpad.reshape(b, _NPIX_PAD, c)
        .transpose(0, 2, 1)
        .reshape(b, 12, 16 * _NPIX_PAD)
    )

    out6 = _sc_sample(img, idx, wts)
    out = out6.transpose(0, 3, 5, 1, 2, 4).reshape(b, _HW, c)
    return out.reshape(b, c, h, w)
